# baseline (speedup 1.0000x reference)
import functools

import jax
import jax.numpy as jnp
from jax import lax
from jax.experimental import pallas as pl
from jax.experimental.pallas import tpu as pltpu
from jax.experimental.pallas import tpu_sc as plsc

_N = 10000
_E = 320000
_D1 = 128
_D2 = 48
_NCLS = 40

_NC = 2
_NS = 16
_NW = _NC * _NS
_NPAD = 10240
_RPT = _NPAD // _NS
_EPT = _E // _NW
_KD = 2000
_K = 200

_mesh = plsc.VectorSubcoreMesh(core_axis_name="c", subcore_axis_name="s")


def _deg_body(src_h, dst_h, ones_h, zed_h, out_h,
              idx_v, ones_v, dego_sp, degi_sp):
    cid = lax.axis_index("c")
    sid = lax.axis_index("s")
    wid = cid * _NS + sid
    r0 = sid * _RPT
    pltpu.sync_copy(zed_h.at[pl.ds(r0, _RPT)], dego_sp.at[pl.ds(r0, _RPT)])
    pltpu.sync_copy(zed_h.at[pl.ds(r0, _RPT)], degi_sp.at[pl.ds(r0, _RPT)])
    pltpu.sync_copy(ones_h, ones_v)
    plsc.subcore_barrier()
    ebase = wid * _EPT

    def body(j, carry):
        b = ebase + j * _KD
        pltpu.sync_copy(src_h.at[pl.ds(b, _KD)], idx_v)
        pltpu.sync_copy(ones_v, dego_sp.at[idx_v], add=True)
        pltpu.sync_copy(dst_h.at[pl.ds(b, _KD)], idx_v)
        pltpu.sync_copy(ones_v, degi_sp.at[idx_v], add=True)
        return carry

    lax.fori_loop(0, _EPT // _KD, body, 0)
    plsc.subcore_barrier()
    pltpu.sync_copy(dego_sp.at[pl.ds(r0, _RPT)],
                    out_h.at[pl.ds(cid * _NPAD + r0, _RPT)])
    pltpu.sync_copy(degi_sp.at[pl.ds(r0, _RPT)],
                    out_h.at[pl.ds((2 + cid) * _NPAD + r0, _RPT)])


_deg_call = pl.kernel(
    _deg_body,
    mesh=_mesh,
    out_type=jax.ShapeDtypeStruct((4 * _NPAD,), jnp.float32),
    scratch_types=[
        pltpu.VMEM((_KD,), jnp.int32),
        pltpu.VMEM((_KD,), jnp.float32),
        pltpu.VMEM_SHARED((_NPAD,), jnp.float32),
        pltpu.VMEM_SHARED((_NPAD,), jnp.float32),
    ],
)


def _seg_body(k, nch, h_h, src3_h, dst3_h, zed_h, out_h,
              sidx_v, didx_v, rows0_v, rows1_v, acc_sp, g0, g1, s0, s1):
    cid = lax.axis_index("c")
    sid = lax.axis_index("s")
    wid = cid * _NS + sid
    r0 = sid * _RPT
    pltpu.sync_copy(zed_h.at[pl.ds(r0, _RPT)], acc_sp.at[pl.ds(r0, _RPT)])
    pltpu.sync_copy(src3_h.at[wid], sidx_v)
    pltpu.sync_copy(dst3_h.at[wid], didx_v)
    plsc.subcore_barrier()

    dummy = h_h.at[pl.ds(0, k)]

    def wait_g(rows_v, sem):
        pltpu.make_async_copy(dummy, rows_v, sem).wait()

    def start_s(rows_v, j, sem):
        pltpu.async_copy(rows_v, acc_sp.at[didx_v.at[j]], sem, add=True)

    def wait_s(rows_v, j, sem):
        pltpu.make_async_copy(rows_v, acc_sp.at[didx_v.at[j]], sem).wait()

    pltpu.async_copy(h_h.at[sidx_v.at[0]], rows0_v, g0)
    wait_g(rows0_v, g0)
    start_s(rows0_v, 0, s0)
    pltpu.async_copy(h_h.at[sidx_v.at[1]], rows1_v, g1)

    def body(i, carry):
        j1 = 2 * i + 1
        wait_g(rows1_v, g1)
        start_s(rows1_v, j1, s1)
        wait_s(rows0_v, j1 - 1, s0)
        pltpu.async_copy(h_h.at[sidx_v.at[j1 + 1]], rows0_v, g0)
        j2 = j1 + 1
        wait_g(rows0_v, g0)
        start_s(rows0_v, j2, s0)
        wait_s(rows1_v, j2 - 1, s1)
        pltpu.async_copy(h_h.at[sidx_v.at[j2 + 1]], rows1_v, g1)
        return carry

    lax.fori_loop(0, (nch - 3) // 2, body, 0)
    wait_g(rows1_v, g1)
    start_s(rows1_v, nch - 2, s1)
    wait_s(rows0_v, nch - 3, s0)
    pltpu.async_copy(h_h.at[sidx_v.at[nch - 1]], rows0_v, g0)
    wait_g(rows0_v, g0)
    start_s(rows0_v, nch - 1, s0)
    wait_s(rows1_v, nch - 2, s1)
    wait_s(rows0_v, nch - 1, s0)
    plsc.subcore_barrier()
    pltpu.sync_copy(acc_sp.at[pl.ds(r0, _RPT)],
                    out_h.at[pl.ds(cid * _NPAD + r0, _RPT)])


def _make_seg(d, k):
    nch = _EPT // k
    assert nch % 2 == 1 and k % 8 == 0
    return pl.kernel(
        functools.partial(_seg_body, k, nch),
        mesh=_mesh,
        out_type=jax.ShapeDtypeStruct((2 * _NPAD, d), jnp.float32),
        scratch_types=[
            pltpu.VMEM((nch, k), jnp.int32),
            pltpu.VMEM((nch, k), jnp.int32),
            pltpu.VMEM((k, d), jnp.float32),
            pltpu.VMEM((k, d), jnp.float32),
            pltpu.VMEM_SHARED((_NPAD, d), jnp.float32),
            pltpu.SemaphoreType.DMA,
            pltpu.SemaphoreType.DMA,
            pltpu.SemaphoreType.DMA,
            pltpu.SemaphoreType.DMA,
        ],
        compiler_params=pltpu.CompilerParams(use_tc_tiling_on_sc=False),
    )


_K1 = 80
_K2 = 400
_seg_d1 = _make_seg(_D1, _K1)
_seg_d2 = _make_seg(_D2, _K2)


def _h1_body(deg_ref, feat_ref, h1_ref):
    d = deg_ref[pl.ds(0, _N), :]
    deg_out = d[:, 0:1] + d[:, 1:2]
    norm_src = lax.rsqrt(jnp.maximum(deg_out, 1.0))
    h1_ref[...] = feat_ref[...] * norm_src


def _mid_body(aggp_ref, deg_ref, w1_ref, b1_ref, w2_ref, h2_ref):
    d = deg_ref[pl.ds(0, _N), :]
    deg_out = d[:, 0:1] + d[:, 1:2]
    deg_in = d[:, 2:3] + d[:, 3:4]
    norm_src = lax.rsqrt(jnp.maximum(deg_out, 1.0))
    norm_dst = lax.rsqrt(jnp.maximum(deg_in, 1.0))
    agg = aggp_ref[pl.ds(0, _N), :] + aggp_ref[pl.ds(_NPAD, _N), :]
    x1 = jnp.dot(agg, w1_ref[...], preferred_element_type=jnp.float32)
    x1 = jnp.maximum(x1 * norm_dst + b1_ref[...][None, :], 0.0)
    h2_ref[...] = jnp.dot(x1 * norm_src, w2_ref[...],
                          preferred_element_type=jnp.float32)


def _fin_body(aggp_ref, deg_ref, b2_ref, out_ref):
    d = deg_ref[pl.ds(0, _N), :]
    deg_in = d[:, 2:3] + d[:, 3:4]
    norm_dst = lax.rsqrt(jnp.maximum(deg_in, 1.0))
    agg = aggp_ref[pl.ds(0, _N), :] + aggp_ref[pl.ds(_NPAD, _N), :]
    out_ref[...] = agg[:, :_NCLS] * norm_dst + b2_ref[...][None, :]


def kernel(feat, edge_index, W1, b1, W2, b2):
    src = edge_index[0].astype(jnp.int32)
    dst = edge_index[1].astype(jnp.int32)

    ones = jnp.ones((_KD,), jnp.float32)
    zed1d = jnp.zeros((_NPAD,), jnp.float32)
    deg4 = _deg_call(src, dst, ones, zed1d)
    degt = jnp.transpose(deg4.reshape(4, _NPAD))

    h1 = pl.pallas_call(
        _h1_body,
        out_shape=jax.ShapeDtypeStruct((_N, _D1), jnp.float32),
    )(degt, feat)

    zed1 = jnp.zeros((_NPAD, _D1), jnp.float32)
    src1 = src.reshape(_NW, _EPT // _K1, _K1)
    dst1 = dst.reshape(_NW, _EPT // _K1, _K1)
    aggp = _seg_d1(h1, src1, dst1, zed1)

    w2p = jnp.pad(W2, ((0, 0), (0, _D2 - _NCLS)))
    h2 = pl.pallas_call(
        _mid_body,
        out_shape=jax.ShapeDtypeStruct((_N, _D2), jnp.float32),
    )(aggp, degt, W1, b1, w2p)

    zed2b = jnp.zeros((_NPAD, _D2), jnp.float32)
    src2 = src.reshape(_NW, _EPT // _K2, _K2)
    dst2 = dst.reshape(_NW, _EPT // _K2, _K2)
    agg2p = _seg_d2(h2, src2, dst2, zed2b)

    out = pl.pallas_call(
        _fin_body,
        out_shape=jax.ShapeDtypeStruct((_N, _NCLS), jnp.float32),
    )(agg2p, degt, b2)
    return out


# speedup vs baseline: 1.1453x; 1.1453x over previous
import functools

import jax
import jax.numpy as jnp
from jax import lax
from jax.experimental import pallas as pl
from jax.experimental.pallas import tpu as pltpu
from jax.experimental.pallas import tpu_sc as plsc

_N = 10000
_E = 320000
_D1 = 128
_D2 = 48
_NCLS = 40

_NC = 2
_NS = 16
_NW = _NC * _NS
_NPAD = 10240
_RPT = _NPAD // _NS
_EPT = _E // _NW
_KD = 2000
_K1 = 80
_K2 = 400
_BLK = 640
_GRID = 16

_mesh = plsc.VectorSubcoreMesh(core_axis_name="c", subcore_axis_name="s")


def _fill(ref, n, value):
    def body(i, carry):
        ref[pl.ds(i * 16, 16)] = jnp.full((16,), value, jnp.float32)
        return carry
    lax.fori_loop(0, n // 16, body, 0)


def _deg_body(src_h, dst_h, out_h, idx_v, ones_v, zed_v, dego_sp, degi_sp):
    cid = lax.axis_index("c")
    sid = lax.axis_index("s")
    wid = cid * _NS + sid
    r0 = sid * _RPT
    _fill(ones_v, _KD, 1.0)
    _fill(zed_v, _RPT, 0.0)
    pltpu.sync_copy(zed_v, dego_sp.at[pl.ds(r0, _RPT)])
    pltpu.sync_copy(zed_v, degi_sp.at[pl.ds(r0, _RPT)])
    plsc.subcore_barrier()
    ebase = wid * _EPT

    def body(j, carry):
        b = ebase + j * _KD
        pltpu.sync_copy(src_h.at[pl.ds(b, _KD)], idx_v)
        pltpu.sync_copy(ones_v, dego_sp.at[idx_v], add=True)
        pltpu.sync_copy(dst_h.at[pl.ds(b, _KD)], idx_v)
        pltpu.sync_copy(ones_v, degi_sp.at[idx_v], add=True)
        return carry

    lax.fori_loop(0, _EPT // _KD, body, 0)
    plsc.subcore_barrier()
    pltpu.sync_copy(dego_sp.at[pl.ds(r0, _RPT)],
                    out_h.at[pl.ds(cid * _NPAD + r0, _RPT)])
    pltpu.sync_copy(degi_sp.at[pl.ds(r0, _RPT)],
                    out_h.at[pl.ds((2 + cid) * _NPAD + r0, _RPT)])


_deg_call = pl.kernel(
    _deg_body,
    mesh=_mesh,
    out_type=jax.ShapeDtypeStruct((4 * _NPAD,), jnp.float32),
    scratch_types=[
        pltpu.VMEM((_KD,), jnp.int32),
        pltpu.VMEM((_KD,), jnp.float32),
        pltpu.VMEM((_RPT,), jnp.float32),
        pltpu.VMEM_SHARED((_NPAD,), jnp.float32),
        pltpu.VMEM_SHARED((_NPAD,), jnp.float32),
    ],
)


def _seg_body(k, nch, d, h_h, src3_h, dst3_h, out_h,
              sidx_v, didx_v, rows0_v, rows1_v, acc_sp, g0, g1):
    cid = lax.axis_index("c")
    sid = lax.axis_index("s")
    wid = cid * _NS + sid
    r0 = sid * _RPT

    def zbody(i, carry):
        for dd in range(d // 16):
            rows0_v[i, pl.ds(dd * 16, 16)] = jnp.zeros((16,), jnp.float32)
        return carry
    lax.fori_loop(0, k, zbody, 0)
    off = 0
    while off < _RPT:
        step = min(k, _RPT - off)
        pltpu.sync_copy(rows0_v.at[pl.ds(0, step)],
                        acc_sp.at[pl.ds(r0 + off, step)])
        off += step

    pltpu.sync_copy(src3_h.at[wid], sidx_v)
    pltpu.sync_copy(dst3_h.at[wid], didx_v)
    plsc.subcore_barrier()

    dummy = h_h.at[pl.ds(0, k)]
    pltpu.async_copy(h_h.at[sidx_v.at[0]], rows0_v, g0)

    def body(i, carry):
        j0 = 2 * i
        pltpu.async_copy(h_h.at[sidx_v.at[j0 + 1]], rows1_v, g1)
        pltpu.make_async_copy(dummy, rows0_v, g0).wait()
        pltpu.sync_copy(rows0_v, acc_sp.at[didx_v.at[j0]], add=True)
        pltpu.async_copy(h_h.at[sidx_v.at[j0 + 2]], rows0_v, g0)
        pltpu.make_async_copy(dummy, rows1_v, g1).wait()
        pltpu.sync_copy(rows1_v, acc_sp.at[didx_v.at[j0 + 1]], add=True)
        return carry

    lax.fori_loop(0, (nch - 1) // 2, body, 0)
    pltpu.make_async_copy(dummy, rows0_v, g0).wait()
    pltpu.sync_copy(rows0_v, acc_sp.at[didx_v.at[nch - 1]], add=True)
    plsc.subcore_barrier()
    pltpu.sync_copy(acc_sp.at[pl.ds(r0, _RPT)],
                    out_h.at[pl.ds(cid * _NPAD + r0, _RPT)])


def _make_seg(d, k):
    nch = _EPT // k
    assert nch % 2 == 1 and k % 8 == 0
    return pl.kernel(
        functools.partial(_seg_body, k, nch, d),
        mesh=_mesh,
        out_type=jax.ShapeDtypeStruct((2 * _NPAD, d), jnp.float32),
        scratch_types=[
            pltpu.VMEM((nch, k), jnp.int32),
            pltpu.VMEM((nch, k), jnp.int32),
            pltpu.VMEM((k, d), jnp.float32),
            pltpu.VMEM((k, d), jnp.float32),
            pltpu.VMEM_SHARED((_NPAD, d), jnp.float32),
            pltpu.SemaphoreType.DMA,
            pltpu.SemaphoreType.DMA,
        ],
        compiler_params=pltpu.CompilerParams(use_tc_tiling_on_sc=False),
    )


_seg_d1 = _make_seg(_D1, _K1)
_seg_d2 = _make_seg(_D2, _K2)


def _h1_body(deg_ref, feat_ref, h1_ref):
    d = deg_ref[...]
    deg_out = d[:, 0:1] + d[:, 1:2]
    norm_src = lax.rsqrt(jnp.maximum(deg_out, 1.0))
    h1_ref[...] = feat_ref[...] * norm_src


def _mid_body(a0_ref, a1_ref, deg_ref, w1_ref, b1_ref, w2_ref, h2_ref):
    d = deg_ref[...]
    deg_out = d[:, 0:1] + d[:, 1:2]
    deg_in = d[:, 2:3] + d[:, 3:4]
    norm_src = lax.rsqrt(jnp.maximum(deg_out, 1.0))
    norm_dst = lax.rsqrt(jnp.maximum(deg_in, 1.0))
    agg = a0_ref[...] + a1_ref[...]
    x1 = jnp.dot(agg, w1_ref[...], preferred_element_type=jnp.float32)
    x1 = jnp.maximum(x1 * norm_dst + b1_ref[...][None, :], 0.0)
    h2_ref[...] = jnp.dot(x1 * norm_src, w2_ref[...],
                          preferred_element_type=jnp.float32)


def _fin_body(a0_ref, a1_ref, deg_ref, b2_ref, out_ref):
    d = deg_ref[...]
    deg_in = d[:, 2:3] + d[:, 3:4]
    norm_dst = lax.rsqrt(jnp.maximum(deg_in, 1.0))
    agg = a0_ref[...] + a1_ref[...]
    out_ref[...] = agg[:, :_NCLS] * norm_dst + b2_ref[...][None, :]


def _row_spec(w, roff=0):
    return pl.BlockSpec((_BLK, w), lambda i, _r=roff: (i + _r, 0))


def _const_spec(shape):
    nd = len(shape)
    if nd == 1:
        return pl.BlockSpec(shape, lambda i: (0,))
    return pl.BlockSpec(shape, lambda i: (0, 0))


def kernel(feat, edge_index, W1, b1, W2, b2):
    src = edge_index[0].astype(jnp.int32)
    dst = edge_index[1].astype(jnp.int32)
    src1 = src.reshape(_NW, _EPT // _K1, _K1)
    dst1 = dst.reshape(_NW, _EPT // _K1, _K1)
    src2 = src.reshape(_NW, _EPT // _K2, _K2)
    dst2 = dst.reshape(_NW, _EPT // _K2, _K2)

    deg4 = _deg_call(src, dst)
    degt = jnp.transpose(deg4.reshape(4, _NPAD))

    h1 = pl.pallas_call(
        _h1_body,
        grid=(_GRID,),
        in_specs=[_row_spec(4), _row_spec(_D1)],
        out_specs=_row_spec(_D1),
        out_shape=jax.ShapeDtypeStruct((_N, _D1), jnp.float32),
    )(degt, feat)

    aggp = _seg_d1(h1, src1, dst1)

    w2p = jnp.pad(W2, ((0, 0), (0, _D2 - _NCLS)))
    h2 = pl.pallas_call(
        _mid_body,
        grid=(_GRID,),
        in_specs=[_row_spec(_D1), _row_spec(_D1, _GRID), _row_spec(4),
                  _const_spec((128, _D1)), _const_spec((_D1,)),
                  _const_spec((_D1, _D2))],
        out_specs=_row_spec(_D2),
        out_shape=jax.ShapeDtypeStruct((_N, _D2), jnp.float32),
    )(aggp, aggp, degt, W1, b1, w2p)

    agg2p = _seg_d2(h2, src2, dst2)

    out = pl.pallas_call(
        _fin_body,
        grid=(_GRID,),
        in_specs=[_row_spec(_D2), _row_spec(_D2, _GRID), _row_spec(4),
                  _const_spec((_NCLS,))],
        out_specs=_row_spec(_NCLS),
        out_shape=jax.ShapeDtypeStruct((_N, _NCLS), jnp.float32),
    )(agg2p, agg2p, degt, b2)
    return out


# speedup vs baseline: 1.2084x; 1.0551x over previous
import functools

import jax
import jax.numpy as jnp
from jax import lax
from jax.experimental import pallas as pl
from jax.experimental.pallas import tpu as pltpu
from jax.experimental.pallas import tpu_sc as plsc

_N = 10000
_E = 320000
_D1 = 128
_D2 = 48
_NCLS = 40

_NC = 2
_NS = 16
_NW = _NC * _NS
_NPAD = 10240
_RPT = _NPAD // _NS
_EPT = _E // _NW
_KD = 2000
_K1 = 80
_K2 = 400
_BLK = 640
_GRID = 16

_mesh = plsc.VectorSubcoreMesh(core_axis_name="c", subcore_axis_name="s")


def _fill(ref, n, value):
    def body(i, carry):
        ref[pl.ds(i * 16, 16)] = jnp.full((16,), value, jnp.float32)
        return carry
    lax.fori_loop(0, n // 16, body, 0)


def _deg_body(src_h, dst_h, out_h, idx_v, ones_v, zed_v, dego_sp, degi_sp):
    cid = lax.axis_index("c")
    sid = lax.axis_index("s")
    wid = cid * _NS + sid
    r0 = sid * _RPT
    _fill(ones_v, _KD, 1.0)
    _fill(zed_v, _RPT, 0.0)
    pltpu.sync_copy(zed_v, dego_sp.at[pl.ds(r0, _RPT)])
    pltpu.sync_copy(zed_v, degi_sp.at[pl.ds(r0, _RPT)])
    plsc.subcore_barrier()
    ebase = wid * _EPT

    def body(j, carry):
        b = ebase + j * _KD
        pltpu.sync_copy(src_h.at[pl.ds(b, _KD)], idx_v)
        pltpu.sync_copy(ones_v, dego_sp.at[idx_v], add=True)
        pltpu.sync_copy(dst_h.at[pl.ds(b, _KD)], idx_v)
        pltpu.sync_copy(ones_v, degi_sp.at[idx_v], add=True)
        return carry

    lax.fori_loop(0, _EPT // _KD, body, 0)
    plsc.subcore_barrier()
    pltpu.sync_copy(dego_sp.at[pl.ds(r0, _RPT)],
                    out_h.at[pl.ds(cid * _NPAD + r0, _RPT)])
    pltpu.sync_copy(degi_sp.at[pl.ds(r0, _RPT)],
                    out_h.at[pl.ds((2 + cid) * _NPAD + r0, _RPT)])


_deg_call = pl.kernel(
    _deg_body,
    mesh=_mesh,
    out_type=jax.ShapeDtypeStruct((4 * _NPAD,), jnp.float32),
    scratch_types=[
        pltpu.VMEM((_KD,), jnp.int32),
        pltpu.VMEM((_KD,), jnp.float32),
        pltpu.VMEM((_RPT,), jnp.float32),
        pltpu.VMEM_SHARED((_NPAD,), jnp.float32),
        pltpu.VMEM_SHARED((_NPAD,), jnp.float32),
    ],
)


def _seg_body(k, nch, d, h_h, src3_h, dst3_h, out_h,
              sidx_v, didx_v, rows0_v, rows1_v, acc_sp, g0, g1):
    cid = lax.axis_index("c")
    sid = lax.axis_index("s")
    wid = cid * _NS + sid
    r0 = sid * _RPT

    def zbody(i, carry):
        for dd in range(d // 16):
            rows0_v[i, pl.ds(dd * 16, 16)] = jnp.zeros((16,), jnp.float32)
        return carry
    lax.fori_loop(0, k, zbody, 0)
    off = 0
    while off < _RPT:
        step = min(k, _RPT - off)
        pltpu.sync_copy(rows0_v.at[pl.ds(0, step)],
                        acc_sp.at[pl.ds(r0 + off, step)])
        off += step

    pltpu.sync_copy(src3_h.at[wid], sidx_v)
    pltpu.sync_copy(dst3_h.at[wid], didx_v)
    plsc.subcore_barrier()

    dummy = h_h.at[pl.ds(0, k)]
    pltpu.async_copy(h_h.at[sidx_v.at[0]], rows0_v, g0)

    def body(i, carry):
        j0 = 2 * i
        pltpu.async_copy(h_h.at[sidx_v.at[j0 + 1]], rows1_v, g1)
        pltpu.make_async_copy(dummy, rows0_v, g0).wait()
        pltpu.sync_copy(rows0_v, acc_sp.at[didx_v.at[j0]], add=True)
        pltpu.async_copy(h_h.at[sidx_v.at[j0 + 2]], rows0_v, g0)
        pltpu.make_async_copy(dummy, rows1_v, g1).wait()
        pltpu.sync_copy(rows1_v, acc_sp.at[didx_v.at[j0 + 1]], add=True)
        return carry

    lax.fori_loop(0, (nch - 1) // 2, body, 0)
    pltpu.make_async_copy(dummy, rows0_v, g0).wait()
    pltpu.sync_copy(rows0_v, acc_sp.at[didx_v.at[nch - 1]], add=True)
    plsc.subcore_barrier()
    pltpu.sync_copy(acc_sp.at[pl.ds(r0, _RPT)],
                    out_h.at[pl.ds(cid * _NPAD + r0, _RPT)])


def _make_seg(d, k):
    nch = _EPT // k
    assert nch % 2 == 1 and k % 8 == 0
    return pl.kernel(
        functools.partial(_seg_body, k, nch, d),
        mesh=_mesh,
        out_type=jax.ShapeDtypeStruct((2 * _NPAD, d), jnp.float32),
        scratch_types=[
            pltpu.VMEM((nch, k), jnp.int32),
            pltpu.VMEM((nch, k), jnp.int32),
            pltpu.VMEM((k, d), jnp.float32),
            pltpu.VMEM((k, d), jnp.float32),
            pltpu.VMEM_SHARED((_NPAD, d), jnp.float32),
            pltpu.SemaphoreType.DMA,
            pltpu.SemaphoreType.DMA,
        ],
        compiler_params=pltpu.CompilerParams(use_tc_tiling_on_sc=False),
    )


_seg_d1 = _make_seg(_D1, _K1)
_seg_d2 = _make_seg(_D2, _K2)


def _h1_body(deg_ref, feat_ref, h1_ref):
    d = deg_ref[pl.ds(0, _N), :]
    deg_out = d[:, 0:1] + d[:, 1:2]
    norm_src = lax.rsqrt(jnp.maximum(deg_out, 1.0))
    h1_ref[...] = feat_ref[...] * norm_src


def _mid_body(aggp_ref, deg_ref, w1_ref, b1_ref, w2_ref, h2_ref):
    d = deg_ref[pl.ds(0, _N), :]
    deg_out = d[:, 0:1] + d[:, 1:2]
    deg_in = d[:, 2:3] + d[:, 3:4]
    norm_src = lax.rsqrt(jnp.maximum(deg_out, 1.0))
    norm_dst = lax.rsqrt(jnp.maximum(deg_in, 1.0))
    agg = aggp_ref[pl.ds(0, _N), :] + aggp_ref[pl.ds(_NPAD, _N), :]
    x1 = jnp.dot(agg, w1_ref[...], preferred_element_type=jnp.float32)
    x1 = jnp.maximum(x1 * norm_dst + b1_ref[...][None, :], 0.0)
    h2_ref[...] = jnp.dot(x1 * norm_src, w2_ref[...],
                          preferred_element_type=jnp.float32)


def _fin_body(aggp_ref, deg_ref, b2_ref, out_ref):
    d = deg_ref[pl.ds(0, _N), :]
    deg_in = d[:, 2:3] + d[:, 3:4]
    norm_dst = lax.rsqrt(jnp.maximum(deg_in, 1.0))
    agg = aggp_ref[pl.ds(0, _N), :] + aggp_ref[pl.ds(_NPAD, _N), :]
    out_ref[...] = agg[:, :_NCLS] * norm_dst + b2_ref[...][None, :]


def kernel(feat, edge_index, W1, b1, W2, b2):
    src = edge_index[0].astype(jnp.int32)
    dst = edge_index[1].astype(jnp.int32)
    src1 = src.reshape(_NW, _EPT // _K1, _K1)
    dst1 = dst.reshape(_NW, _EPT // _K1, _K1)
    src2 = src.reshape(_NW, _EPT // _K2, _K2)
    dst2 = dst.reshape(_NW, _EPT // _K2, _K2)

    deg4 = _deg_call(src, dst)
    degt = jnp.transpose(deg4.reshape(4, _NPAD))

    h1 = pl.pallas_call(
        _h1_body,
        out_shape=jax.ShapeDtypeStruct((_N, _D1), jnp.float32),
    )(degt, feat)

    aggp = _seg_d1(h1, src1, dst1)

    w2p = jnp.pad(W2, ((0, 0), (0, _D2 - _NCLS)))
    h2 = pl.pallas_call(
        _mid_body,
        out_shape=jax.ShapeDtypeStruct((_N, _D2), jnp.float32),
    )(aggp, degt, W1, b1, w2p)

    agg2p = _seg_d2(h2, src2, dst2)

    out = pl.pallas_call(
        _fin_body,
        out_shape=jax.ShapeDtypeStruct((_N, _NCLS), jnp.float32),
    )(agg2p, degt, b2)
    return out


# speedup vs baseline: 1.2209x; 1.0103x over previous
import functools

import jax
import jax.numpy as jnp
from jax import lax
from jax.experimental import pallas as pl
from jax.experimental.pallas import tpu as pltpu
from jax.experimental.pallas import tpu_sc as plsc

_N = 10000
_E = 320000
_D1 = 128
_D2 = 48
_NCLS = 40

_NC = 2
_NS = 16
_NW = _NC * _NS
_NPAD = 10240
_RPT = _NPAD // _NS
_EPT = _E // _NW
_KD = 2000
_K1 = 80
_K2 = 400
_BLK = 640
_GRID = 16

_mesh = plsc.VectorSubcoreMesh(core_axis_name="c", subcore_axis_name="s")


def _fill(ref, n, value):
    def body(i, carry):
        ref[pl.ds(i * 16, 16)] = jnp.full((16,), value, jnp.float32)
        return carry
    lax.fori_loop(0, n // 16, body, 0)


def _deg_body(src2_h, dst2_h, out_h, idx_v, ones_v, zed_v, dego_sp, degi_sp):
    cid = lax.axis_index("c")
    sid = lax.axis_index("s")
    wid = cid * _NS + sid
    r0 = sid * _RPT
    _fill(ones_v, _EPT, 1.0)
    _fill(zed_v, _RPT, 0.0)
    pltpu.sync_copy(zed_v, dego_sp.at[pl.ds(r0, _RPT)])
    pltpu.sync_copy(zed_v, degi_sp.at[pl.ds(r0, _RPT)])
    plsc.subcore_barrier()

    pltpu.sync_copy(src2_h.at[wid], idx_v)
    pltpu.sync_copy(ones_v, dego_sp.at[idx_v], add=True)
    pltpu.sync_copy(dst2_h.at[wid], idx_v)
    pltpu.sync_copy(ones_v, degi_sp.at[idx_v], add=True)
    plsc.subcore_barrier()
    pltpu.sync_copy(dego_sp.at[pl.ds(r0, _RPT)],
                    out_h.at[pl.ds(cid * _NPAD + r0, _RPT)])
    pltpu.sync_copy(degi_sp.at[pl.ds(r0, _RPT)],
                    out_h.at[pl.ds((2 + cid) * _NPAD + r0, _RPT)])


_deg_call = pl.kernel(
    _deg_body,
    mesh=_mesh,
    out_type=jax.ShapeDtypeStruct((4 * _NPAD,), jnp.float32),
    scratch_types=[
        pltpu.VMEM((_EPT,), jnp.int32),
        pltpu.VMEM((_EPT,), jnp.float32),
        pltpu.VMEM((_RPT,), jnp.float32),
        pltpu.VMEM_SHARED((_NPAD,), jnp.float32),
        pltpu.VMEM_SHARED((_NPAD,), jnp.float32),
    ],
    compiler_params=pltpu.CompilerParams(use_tc_tiling_on_sc=False),
)


def _seg_body(k, nch, d, h_h, src3_h, dst3_h, out_h,
              sidx_v, didx_v, rows0_v, rows1_v, acc_sp, g0, g1):
    cid = lax.axis_index("c")
    sid = lax.axis_index("s")
    wid = cid * _NS + sid
    r0 = sid * _RPT

    def zbody(i, carry):
        for dd in range(d // 16):
            rows0_v[i, pl.ds(dd * 16, 16)] = jnp.zeros((16,), jnp.float32)
        return carry
    lax.fori_loop(0, k, zbody, 0)
    off = 0
    while off < _RPT:
        step = min(k, _RPT - off)
        pltpu.sync_copy(rows0_v.at[pl.ds(0, step)],
                        acc_sp.at[pl.ds(r0 + off, step)])
        off += step

    pltpu.sync_copy(src3_h.at[wid], sidx_v)
    pltpu.sync_copy(dst3_h.at[wid], didx_v)
    plsc.subcore_barrier()

    dummy = h_h.at[pl.ds(0, k)]
    pltpu.async_copy(h_h.at[sidx_v.at[0]], rows0_v, g0)

    def body(i, carry):
        j0 = 2 * i
        pltpu.async_copy(h_h.at[sidx_v.at[j0 + 1]], rows1_v, g1)
        pltpu.make_async_copy(dummy, rows0_v, g0).wait()
        pltpu.sync_copy(rows0_v, acc_sp.at[didx_v.at[j0]], add=True)
        pltpu.async_copy(h_h.at[sidx_v.at[j0 + 2]], rows0_v, g0)
        pltpu.make_async_copy(dummy, rows1_v, g1).wait()
        pltpu.sync_copy(rows1_v, acc_sp.at[didx_v.at[j0 + 1]], add=True)
        return carry

    lax.fori_loop(0, (nch - 1) // 2, body, 0)
    pltpu.make_async_copy(dummy, rows0_v, g0).wait()
    pltpu.sync_copy(rows0_v, acc_sp.at[didx_v.at[nch - 1]], add=True)
    plsc.subcore_barrier()
    pltpu.sync_copy(acc_sp.at[pl.ds(r0, _RPT)],
                    out_h.at[pl.ds(cid * _NPAD + r0, _RPT)])


def _make_seg(d, k):
    nch = _EPT // k
    assert nch % 2 == 1 and k % 8 == 0
    return pl.kernel(
        functools.partial(_seg_body, k, nch, d),
        mesh=_mesh,
        out_type=jax.ShapeDtypeStruct((2 * _NPAD, d), jnp.float32),
        scratch_types=[
            pltpu.VMEM((nch, k), jnp.int32),
            pltpu.VMEM((nch, k), jnp.int32),
            pltpu.VMEM((k, d), jnp.float32),
            pltpu.VMEM((k, d), jnp.float32),
            pltpu.VMEM_SHARED((_NPAD, d), jnp.float32),
            pltpu.SemaphoreType.DMA,
            pltpu.SemaphoreType.DMA,
        ],
        compiler_params=pltpu.CompilerParams(use_tc_tiling_on_sc=False),
    )


_seg_d1 = _make_seg(_D1, _K1)
_seg_d2 = _make_seg(_D2, _K2)


def _h1_body(deg_ref, feat_ref, h1_ref):
    d = deg_ref[pl.ds(0, _N), :]
    deg_out = d[:, 0:1] + d[:, 1:2]
    norm_src = lax.rsqrt(jnp.maximum(deg_out, 1.0))
    h1_ref[...] = feat_ref[...] * norm_src


def _mid_body(aggp_ref, deg_ref, w1_ref, b1_ref, w2_ref, h2_ref):
    d = deg_ref[pl.ds(0, _N), :]
    deg_out = d[:, 0:1] + d[:, 1:2]
    deg_in = d[:, 2:3] + d[:, 3:4]
    norm_src = lax.rsqrt(jnp.maximum(deg_out, 1.0))
    norm_dst = lax.rsqrt(jnp.maximum(deg_in, 1.0))
    agg = aggp_ref[pl.ds(0, _N), :] + aggp_ref[pl.ds(_NPAD, _N), :]
    x1 = jnp.dot(agg, w1_ref[...], preferred_element_type=jnp.float32)
    x1 = jnp.maximum(x1 * norm_dst + b1_ref[...][None, :], 0.0)
    h2_ref[...] = jnp.dot(x1 * norm_src, w2_ref[...],
                          preferred_element_type=jnp.float32)


def _fin_body(aggp_ref, deg_ref, b2_ref, out_ref):
    d = deg_ref[pl.ds(0, _N), :]
    deg_in = d[:, 2:3] + d[:, 3:4]
    norm_dst = lax.rsqrt(jnp.maximum(deg_in, 1.0))
    agg = aggp_ref[pl.ds(0, _N), :] + aggp_ref[pl.ds(_NPAD, _N), :]
    out_ref[...] = agg[:, :_NCLS] * norm_dst + b2_ref[...][None, :]


def kernel(feat, edge_index, W1, b1, W2, b2):
    src = edge_index[0].astype(jnp.int32)
    dst = edge_index[1].astype(jnp.int32)
    src1 = src.reshape(_NW, _EPT // _K1, _K1)
    dst1 = dst.reshape(_NW, _EPT // _K1, _K1)
    src2 = src.reshape(_NW, _EPT // _K2, _K2)
    dst2 = dst.reshape(_NW, _EPT // _K2, _K2)

    deg4 = _deg_call(src.reshape(_NW, _EPT), dst.reshape(_NW, _EPT))
    degt = jnp.transpose(deg4.reshape(4, _NPAD))

    h1 = pl.pallas_call(
        _h1_body,
        out_shape=jax.ShapeDtypeStruct((_N, _D1), jnp.float32),
    )(degt, feat)

    aggp = _seg_d1(h1, src1, dst1)

    w2p = jnp.pad(W2, ((0, 0), (0, _D2 - _NCLS)))
    h2 = pl.pallas_call(
        _mid_body,
        out_shape=jax.ShapeDtypeStruct((_N, _D2), jnp.float32),
    )(aggp, degt, W1, b1, w2p)

    agg2p = _seg_d2(h2, src2, dst2)

    out = pl.pallas_call(
        _fin_body,
        out_shape=jax.ShapeDtypeStruct((_N, _NCLS), jnp.float32),
    )(agg2p, degt, b2)
    return out


# speedup vs baseline: 1.2641x; 1.0354x over previous
import functools

import jax
import jax.numpy as jnp
from jax import lax
from jax.experimental import pallas as pl
from jax.experimental.pallas import tpu as pltpu
from jax.experimental.pallas import tpu_sc as plsc

_N = 10000
_E = 320000
_D1 = 128
_D2 = 48
_NCLS = 40

_NC = 2
_NS = 16
_NW = _NC * _NS
_NPAD = 10240
_RPT = _NPAD // _NS
_EPT = _E // _NW
_KD = 2000
_K1 = 80
_K2 = 400
_BLK = 640
_GRID = 16

_mesh = plsc.VectorSubcoreMesh(core_axis_name="c", subcore_axis_name="s")


def _fill(ref, n, value):
    def body(i, carry):
        ref[pl.ds(i * 16, 16)] = jnp.full((16,), value, jnp.float32)
        return carry
    lax.fori_loop(0, n // 16, body, 0)


def _deg_body(src2_h, dst2_h, out_h, idx_v, ones_v, zed_v, dego_sp, degi_sp):
    cid = lax.axis_index("c")
    sid = lax.axis_index("s")
    wid = cid * _NS + sid
    r0 = sid * _RPT
    _fill(ones_v, _EPT, 1.0)
    _fill(zed_v, _RPT, 0.0)
    pltpu.sync_copy(zed_v, dego_sp.at[pl.ds(r0, _RPT)])
    pltpu.sync_copy(zed_v, degi_sp.at[pl.ds(r0, _RPT)])
    plsc.subcore_barrier()

    pltpu.sync_copy(src2_h.at[wid], idx_v)
    pltpu.sync_copy(ones_v, dego_sp.at[idx_v], add=True)
    pltpu.sync_copy(dst2_h.at[wid], idx_v)
    pltpu.sync_copy(ones_v, degi_sp.at[idx_v], add=True)
    plsc.subcore_barrier()
    pltpu.sync_copy(dego_sp.at[pl.ds(r0, _RPT)],
                    out_h.at[pl.ds(cid * _NPAD + r0, _RPT)])
    pltpu.sync_copy(degi_sp.at[pl.ds(r0, _RPT)],
                    out_h.at[pl.ds((2 + cid) * _NPAD + r0, _RPT)])


_deg_call = pl.kernel(
    _deg_body,
    mesh=_mesh,
    out_type=jax.ShapeDtypeStruct((4 * _NPAD,), jnp.float32),
    scratch_types=[
        pltpu.VMEM((_EPT,), jnp.int32),
        pltpu.VMEM((_EPT,), jnp.float32),
        pltpu.VMEM((_RPT,), jnp.float32),
        pltpu.VMEM_SHARED((_NPAD,), jnp.float32),
        pltpu.VMEM_SHARED((_NPAD,), jnp.float32),
    ],
    compiler_params=pltpu.CompilerParams(use_tc_tiling_on_sc=False),
)


def _seg_body(k, nch, d, h_h, src3_h, dst3_h, out_h,
              sidx_v, didx_v, rows0_v, rows1_v, acc_sp, g0, g1):
    cid = lax.axis_index("c")
    sid = lax.axis_index("s")
    wid = cid * _NS + sid
    r0 = sid * _RPT

    def zbody(i, carry):
        for dd in range(d // 16):
            rows0_v[i, pl.ds(dd * 16, 16)] = jnp.zeros((16,), jnp.float32)
        return carry
    lax.fori_loop(0, k, zbody, 0)
    off = 0
    while off < _RPT:
        step = min(k, _RPT - off)
        pltpu.sync_copy(rows0_v.at[pl.ds(0, step)],
                        acc_sp.at[pl.ds(r0 + off, step)])
        off += step

    pltpu.sync_copy(src3_h.at[wid], sidx_v)
    pltpu.sync_copy(dst3_h.at[wid], didx_v)
    plsc.subcore_barrier()

    dummy = h_h.at[pl.ds(0, k)]
    pltpu.async_copy(h_h.at[sidx_v.at[0]], rows0_v, g0)

    def body(i, carry):
        j0 = 2 * i
        pltpu.async_copy(h_h.at[sidx_v.at[j0 + 1]], rows1_v, g1)
        pltpu.make_async_copy(dummy, rows0_v, g0).wait()
        pltpu.sync_copy(rows0_v, acc_sp.at[didx_v.at[j0]], add=True)
        pltpu.async_copy(h_h.at[sidx_v.at[j0 + 2]], rows0_v, g0)
        pltpu.make_async_copy(dummy, rows1_v, g1).wait()
        pltpu.sync_copy(rows1_v, acc_sp.at[didx_v.at[j0 + 1]], add=True)
        return carry

    lax.fori_loop(0, (nch - 1) // 2, body, 0)
    pltpu.make_async_copy(dummy, rows0_v, g0).wait()
    pltpu.sync_copy(rows0_v, acc_sp.at[didx_v.at[nch - 1]], add=True)
    plsc.subcore_barrier()
    pltpu.sync_copy(acc_sp.at[pl.ds(r0, _RPT)],
                    out_h.at[pl.ds(cid * _NPAD + r0, _RPT)])


def _make_seg(d, k):
    nch = _EPT // k
    assert nch % 2 == 1 and k % 8 == 0
    return pl.kernel(
        functools.partial(_seg_body, k, nch, d),
        mesh=_mesh,
        out_type=jax.ShapeDtypeStruct((2 * _NPAD, d), jnp.float32),
        scratch_types=[
            pltpu.VMEM((nch, k), jnp.int32),
            pltpu.VMEM((nch, k), jnp.int32),
            pltpu.VMEM((k, d), jnp.float32),
            pltpu.VMEM((k, d), jnp.float32),
            pltpu.VMEM_SHARED((_NPAD, d), jnp.float32),
            pltpu.SemaphoreType.DMA,
            pltpu.SemaphoreType.DMA,
        ],
        compiler_params=pltpu.CompilerParams(use_tc_tiling_on_sc=False),
    )


_seg_d1 = _make_seg(_D1, _K1)
_seg_d2 = _make_seg(_D2, _K2)


def _h1_body(deg_ref, feat_ref, h1_ref):
    d = jnp.transpose(deg_ref[...])[:_N]
    deg_out = d[:, 0:1] + d[:, 1:2]
    norm_src = lax.rsqrt(jnp.maximum(deg_out, 1.0))
    h1_ref[...] = feat_ref[...] * norm_src


def _mid_body(aggp_ref, deg_ref, w1_ref, b1_ref, w2_ref, h2_ref):
    d = jnp.transpose(deg_ref[...])[:_N]
    deg_out = d[:, 0:1] + d[:, 1:2]
    deg_in = d[:, 2:3] + d[:, 3:4]
    norm_src = lax.rsqrt(jnp.maximum(deg_out, 1.0))
    norm_dst = lax.rsqrt(jnp.maximum(deg_in, 1.0))
    agg = aggp_ref[pl.ds(0, _N), :] + aggp_ref[pl.ds(_NPAD, _N), :]
    x1 = jnp.dot(agg, w1_ref[...], preferred_element_type=jnp.float32)
    x1 = jnp.maximum(x1 * norm_dst + b1_ref[...][None, :], 0.0)
    h2_ref[...] = jnp.dot(x1 * norm_src, w2_ref[...],
                          preferred_element_type=jnp.float32)


def _fin_body(aggp_ref, deg_ref, b2_ref, out_ref):
    d = jnp.transpose(deg_ref[...])[:_N]
    deg_in = d[:, 2:3] + d[:, 3:4]
    norm_dst = lax.rsqrt(jnp.maximum(deg_in, 1.0))
    agg = aggp_ref[pl.ds(0, _N), :] + aggp_ref[pl.ds(_NPAD, _N), :]
    res = agg[:, :_NCLS] * norm_dst + b2_ref[...][None, :]
    out_ref[...] = jnp.transpose(res)


def kernel(feat, edge_index, W1, b1, W2, b2):
    src = edge_index[0].astype(jnp.int32)
    dst = edge_index[1].astype(jnp.int32)
    src1 = src.reshape(_NW, _EPT // _K1, _K1)
    dst1 = dst.reshape(_NW, _EPT // _K1, _K1)
    src2 = src.reshape(_NW, _EPT // _K2, _K2)
    dst2 = dst.reshape(_NW, _EPT // _K2, _K2)

    deg4 = _deg_call(src.reshape(_NW, _EPT), dst.reshape(_NW, _EPT))
    degt = deg4.reshape(4, _NPAD)

    h1 = pl.pallas_call(
        _h1_body,
        out_shape=jax.ShapeDtypeStruct((_N, _D1), jnp.float32),
    )(degt, feat)

    aggp = _seg_d1(h1, src1, dst1)

    w2p = jnp.pad(W2, ((0, 0), (0, _D2 - _NCLS)))
    h2 = pl.pallas_call(
        _mid_body,
        out_shape=jax.ShapeDtypeStruct((_N, _D2), jnp.float32),
    )(aggp, degt, W1, b1, w2p)

    agg2p = _seg_d2(h2, src2, dst2)

    out_t = pl.pallas_call(
        _fin_body,
        out_shape=jax.ShapeDtypeStruct((_NCLS, _N), jnp.float32),
    )(agg2p, degt, b2)
    return jnp.transpose(out_t)
